# baseline (speedup 1.0000x reference)
import functools

import jax
import jax.numpy as jnp
from jax import lax
from jax.experimental import pallas as pl
from jax.experimental.pallas import tpu as pltpu
from jax.experimental.pallas import tpu_sc as plsc

B = 16
L = 2048
D = 128
T = B * L
NC = 2
NS = 16
LANES = 16
NVREG = D // LANES
ROWS_PER_W = T // (NC * NS)
CH = 256
NCHUNK = ROWS_PER_W // CH
ZROW = 2 * CH
SLOT = D + LANES

_mesh = plsc.VectorSubcoreMesh(core_axis_name="c", subcore_axis_name="s")


@functools.partial(
    pl.kernel,
    out_type=jax.ShapeDtypeStruct((B, D), jnp.float32),
    mesh=_mesh,
    compiler_params=pltpu.CompilerParams(needs_layout_passes=False,
                                         use_tc_tiling_on_sc=False),
    scratch_types=[
        pltpu.VMEM((2 * CH + 1, D), jnp.float32),
        pltpu.VMEM((ROWS_PER_W,), jnp.int32),
        pltpu.VMEM((CH + LANES,), jnp.int32),
        pltpu.VMEM((SLOT,), jnp.float32),
        pltpu.VMEM((SLOT,), jnp.float32),
        pltpu.VMEM((SLOT,), jnp.float32),
        pltpu.VMEM_SHARED((NS, SLOT), jnp.float32),
        pltpu.SemaphoreType.DMA,
        pltpu.SemaphoreType.DMA,
    ],
)
def _readout(emb_hbm, op_hbm, out_hbm, emb_buf, op_buf, corr_buf,
             acc_buf, buf_a, buf_b, shared, sem0, sem1):
    c = lax.axis_index("c")
    s = lax.axis_index("s")
    graph = c * (B // NC) + s // 2
    half = s % 2
    row0 = graph * L + half * ROWS_PER_W

    pltpu.sync_copy(op_hbm.at[graph, pl.ds(half * ROWS_PER_W, ROWS_PER_W)],
                    op_buf)

    sems = [sem0, sem1]
    pltpu.async_copy(emb_hbm.at[pl.ds(row0, CH)], emb_buf.at[pl.ds(0, CH)],
                     sem0)

    zero = jnp.zeros((LANES,), jnp.float32)
    for j in range(NVREG):
        emb_buf[ZROW, pl.ds(j * LANES, LANES)] = zero

    def chunk_body(k, carry):
        cnt5v = carry[NVREG]
        par = k % 2
        boff = par * CH

        for p in range(2):
            @pl.when((k + 1 < NCHUNK) & (par == p))
            def _start(p=p):
                pltpu.async_copy(
                    emb_hbm.at[pl.ds(row0 + (k + 1) * CH, CH)],
                    emb_buf.at[pl.ds((1 - p) * CH, CH)], sems[1 - p])

        def build_body(g, bc):
            base, cv = bc
            opv = op_buf[pl.ds(k * CH + g * LANES, LANES)]
            m5 = opv == 5
            idxv = boff + g * LANES + lax.iota(jnp.int32, LANES)
            plsc.store_compressed(corr_buf.at[pl.ds(base, LANES)], idxv,
                                  mask=m5)
            pc = plsc.all_reduce_population_count(m5)
            return base + pc[0], cv + pc

        base, cnt5v = lax.fori_loop(0, CH // LANES, build_body,
                                    (jnp.int32(0), cnt5v))
        corr_buf[pl.ds(base, LANES)] = jnp.full((LANES,), ZROW, jnp.int32)

        for p in range(2):
            @pl.when(par == p)
            def _wait(p=p):
                pltpu.make_async_copy(
                    emb_hbm.at[pl.ds(row0, CH)],
                    emb_buf.at[pl.ds(p * CH, CH)], sems[p]).wait()

        def grp_body(g, gacc):
            accs = list(gacc)
            r0 = boff + g * LANES
            for i in range(LANES):
                for j in range(NVREG):
                    row = emb_buf[r0 + i, pl.ds(j * LANES, LANES)]
                    accs[j] = accs[j] + row
            return tuple(accs)

        gacc = lax.fori_loop(0, CH // LANES, grp_body, carry[:NVREG],
                             unroll=2)

        def corr_body(t, cacc):
            accs = list(cacc)
            idxv = corr_buf[pl.ds(t * LANES, LANES)]
            for i in range(LANES):
                ri = idxv[i]
                for j in range(NVREG):
                    row = emb_buf[ri, pl.ds(j * LANES, LANES)]
                    accs[j] = accs[j] - row
            return tuple(accs)

        gacc = lax.fori_loop(0, (base + LANES - 1) // LANES, corr_body, gacc)
        return gacc + (cnt5v,)

    init = tuple(jnp.zeros((LANES,), jnp.float32) for _ in range(NVREG))
    fin = lax.fori_loop(0, NCHUNK, chunk_body,
                        init + (jnp.zeros((LANES,), jnp.int32),))
    cnt5v = fin[NVREG]

    for j in range(NVREG):
        acc_buf[pl.ds(j * LANES, LANES)] = fin[j]
    acc_buf[pl.ds(D, LANES)] = (ROWS_PER_W - cnt5v).astype(jnp.float32)

    pltpu.sync_copy(acc_buf, shared.at[s])
    plsc.subcore_barrier()

    @pl.when(half == 0)
    def _combine():
        pltpu.sync_copy(shared.at[s], buf_a)
        pltpu.sync_copy(shared.at[s + 1], buf_b)
        cnt = buf_a[pl.ds(D, LANES)] + buf_b[pl.ds(D, LANES)]
        for j in range(NVREG):
            tot = (buf_a[pl.ds(j * LANES, LANES)] +
                   buf_b[pl.ds(j * LANES, LANES)])
            acc_buf[pl.ds(j * LANES, LANES)] = tot / cnt
        pltpu.sync_copy(acc_buf.at[pl.ds(0, D)], out_hbm.at[graph])


def kernel(node_embeddings, op_idx):
    return _readout(node_embeddings, op_idx.astype(jnp.int32))


# speedup vs baseline: 1.2003x; 1.2003x over previous
import functools

import jax
import jax.numpy as jnp
from jax import lax
from jax.experimental import pallas as pl
from jax.experimental.pallas import tpu as pltpu
from jax.experimental.pallas import tpu_sc as plsc

B = 16
L = 2048
D = 128
T = B * L
NC = 2
NS = 16
LANES = 16
NVREG = D // LANES
ROWS_PER_W = T // (NC * NS)
CH = 256
NCHUNK = ROWS_PER_W // CH
SLOT = D + LANES

_mesh = plsc.VectorSubcoreMesh(core_axis_name="c", subcore_axis_name="s")


@functools.partial(
    pl.kernel,
    out_type=jax.ShapeDtypeStruct((B, D), jnp.float32),
    mesh=_mesh,
    compiler_params=pltpu.CompilerParams(needs_layout_passes=False,
                                         use_tc_tiling_on_sc=False),
    scratch_types=[
        pltpu.VMEM((2 * CH, D), jnp.float32),
        pltpu.VMEM((ROWS_PER_W,), jnp.int32),
        pltpu.VMEM((SLOT,), jnp.float32),
        pltpu.VMEM((SLOT,), jnp.float32),
        pltpu.VMEM((SLOT,), jnp.float32),
        pltpu.VMEM_SHARED((NS, SLOT), jnp.float32),
        pltpu.SemaphoreType.DMA,
        pltpu.SemaphoreType.DMA,
    ],
)
def _readout(emb_hbm, op_hbm, out_hbm, emb_buf, op_buf, acc_buf, buf_a,
             buf_b, shared, sem0, sem1):
    c = lax.axis_index("c")
    s = lax.axis_index("s")
    graph = c * (B // NC) + s // 2
    half = s % 2
    row0 = graph * L + half * ROWS_PER_W

    pltpu.sync_copy(op_hbm.at[graph, pl.ds(half * ROWS_PER_W, ROWS_PER_W)],
                    op_buf)

    sems = [sem0, sem1]
    pltpu.async_copy(emb_hbm.at[pl.ds(row0, CH)], emb_buf.at[pl.ds(0, CH)],
                     sem0)

    def cnt_body(g, cv):
        opv = op_buf[pl.ds(g * LANES, LANES)]
        return cv + plsc.all_reduce_population_count(opv != 5)

    cntv = lax.fori_loop(0, ROWS_PER_W // LANES, cnt_body,
                         jnp.zeros((LANES,), jnp.int32))

    def chunk_body(k, carry):
        par = k % 2
        boff = par * CH

        for p in range(2):
            @pl.when((k + 1 < NCHUNK) & (par == p))
            def _start(p=p):
                pltpu.async_copy(
                    emb_hbm.at[pl.ds(row0 + (k + 1) * CH, CH)],
                    emb_buf.at[pl.ds((1 - p) * CH, CH)], sems[1 - p])

        for p in range(2):
            @pl.when(par == p)
            def _wait(p=p):
                pltpu.make_async_copy(
                    emb_hbm.at[pl.ds(row0, CH)],
                    emb_buf.at[pl.ds(p * CH, CH)], sems[p]).wait()

        def grp_body(g, gcarry):
            acc = list(gcarry)
            opv = op_buf[pl.ds(k * CH + g * LANES, LANES)]
            maskv = jnp.where(opv != 5, 1.0, 0.0).astype(jnp.float32)
            r0 = boff + g * LANES
            for i in range(LANES):
                mf = maskv[i]
                for j in range(NVREG):
                    row = emb_buf[r0 + i, pl.ds(j * LANES, LANES)]
                    acc[j] = acc[j] + row * mf
            return tuple(acc)

        return lax.fori_loop(0, CH // LANES, grp_body, carry)

    init = tuple(jnp.zeros((LANES,), jnp.float32) for _ in range(NVREG))
    fin = lax.fori_loop(0, NCHUNK, chunk_body, init)
    accs = list(fin)

    for j in range(NVREG):
        acc_buf[pl.ds(j * LANES, LANES)] = accs[j]
    acc_buf[pl.ds(D, LANES)] = cntv.astype(jnp.float32)

    pltpu.sync_copy(acc_buf, shared.at[s])
    plsc.subcore_barrier()

    @pl.when(half == 0)
    def _combine():
        pltpu.sync_copy(shared.at[s], buf_a)
        pltpu.sync_copy(shared.at[s + 1], buf_b)
        cnt = buf_a[pl.ds(D, LANES)] + buf_b[pl.ds(D, LANES)]
        for j in range(NVREG):
            tot = (buf_a[pl.ds(j * LANES, LANES)] +
                   buf_b[pl.ds(j * LANES, LANES)])
            acc_buf[pl.ds(j * LANES, LANES)] = tot / cnt
        pltpu.sync_copy(acc_buf.at[pl.ds(0, D)], out_hbm.at[graph])


def kernel(node_embeddings, op_idx):
    return _readout(node_embeddings, op_idx.astype(jnp.int32))


# speedup vs baseline: 1.2003x; 1.0001x over previous
import functools

import jax
import jax.numpy as jnp
from jax import lax
from jax.experimental import pallas as pl
from jax.experimental.pallas import tpu as pltpu
from jax.experimental.pallas import tpu_sc as plsc

B = 16
L = 2048
D = 128
T = B * L
NC = 2
NS = 16
LANES = 16
NVREG = D // LANES
ROWS_PER_W = T // (NC * NS)
CH = 256
NCHUNK = ROWS_PER_W // CH
SLOT = D + LANES

_mesh = plsc.VectorSubcoreMesh(core_axis_name="c", subcore_axis_name="s")


@functools.partial(
    pl.kernel,
    out_type=jax.ShapeDtypeStruct((B, D), jnp.float32),
    mesh=_mesh,
    compiler_params=pltpu.CompilerParams(needs_layout_passes=False,
                                         use_tc_tiling_on_sc=False),
    scratch_types=[
        pltpu.VMEM((2 * CH, D), jnp.float32),
        pltpu.VMEM((ROWS_PER_W,), jnp.int32),
        pltpu.VMEM((SLOT,), jnp.float32),
        pltpu.VMEM((SLOT,), jnp.float32),
        pltpu.VMEM((SLOT,), jnp.float32),
        pltpu.VMEM_SHARED((NS, SLOT), jnp.float32),
        pltpu.SemaphoreType.DMA,
        pltpu.SemaphoreType.DMA,
    ],
)
def _readout(emb_hbm, op_hbm, out_hbm, emb_buf, op_buf, acc_buf, buf_a,
             buf_b, shared, sem0, sem1):
    c = lax.axis_index("c")
    s = lax.axis_index("s")
    graph = c * (B // NC) + s // 2
    half = s % 2
    row0 = graph * L + half * ROWS_PER_W

    pltpu.sync_copy(op_hbm.at[graph, pl.ds(half * ROWS_PER_W, ROWS_PER_W)],
                    op_buf)

    sems = [sem0, sem1]
    pltpu.async_copy(emb_hbm.at[pl.ds(row0, CH)], emb_buf.at[pl.ds(0, CH)],
                     sem0)

    def cnt_body(g, cv):
        opv = op_buf[pl.ds(g * LANES, LANES)]
        return cv + plsc.all_reduce_population_count(opv != 5)

    cntv = lax.fori_loop(0, ROWS_PER_W // LANES, cnt_body,
                         jnp.zeros((LANES,), jnp.int32))

    def chunk_body(k, carry):
        par = k % 2
        boff = par * CH

        for p in range(2):
            @pl.when((k + 1 < NCHUNK) & (par == p))
            def _start(p=p):
                pltpu.async_copy(
                    emb_hbm.at[pl.ds(row0 + (k + 1) * CH, CH)],
                    emb_buf.at[pl.ds((1 - p) * CH, CH)], sems[1 - p])

        for p in range(2):
            @pl.when(par == p)
            def _wait(p=p):
                pltpu.make_async_copy(
                    emb_hbm.at[pl.ds(row0, CH)],
                    emb_buf.at[pl.ds(p * CH, CH)], sems[p]).wait()

        def grp_body(g, gcarry):
            acc = list(gcarry)
            opv = op_buf[pl.ds(k * CH + g * LANES, LANES)]
            maskv = jnp.where(opv != 5, 1.0, 0.0).astype(jnp.float32)
            r0 = boff + g * LANES
            mvs = [jnp.full((LANES,), maskv[i], jnp.float32)
                   for i in range(LANES)]
            for i in range(LANES):
                for j in range(NVREG):
                    row = emb_buf[r0 + i, pl.ds(j * LANES, LANES)]
                    acc[j] = acc[j] + row * mvs[i]
            return tuple(acc)

        return lax.fori_loop(0, CH // LANES, grp_body, carry)

    init = tuple(jnp.zeros((LANES,), jnp.float32) for _ in range(NVREG))
    fin = lax.fori_loop(0, NCHUNK, chunk_body, init)
    accs = list(fin)

    for j in range(NVREG):
        acc_buf[pl.ds(j * LANES, LANES)] = accs[j]
    acc_buf[pl.ds(D, LANES)] = cntv.astype(jnp.float32)

    pltpu.sync_copy(acc_buf, shared.at[s])
    plsc.subcore_barrier()

    @pl.when(half == 0)
    def _combine():
        pltpu.sync_copy(shared.at[s], buf_a)
        pltpu.sync_copy(shared.at[s + 1], buf_b)
        cnt = buf_a[pl.ds(D, LANES)] + buf_b[pl.ds(D, LANES)]
        for j in range(NVREG):
            tot = (buf_a[pl.ds(j * LANES, LANES)] +
                   buf_b[pl.ds(j * LANES, LANES)])
            acc_buf[pl.ds(j * LANES, LANES)] = tot / cnt
        pltpu.sync_copy(acc_buf.at[pl.ds(0, D)], out_hbm.at[graph])


def kernel(node_embeddings, op_idx):
    return _readout(node_embeddings, op_idx.astype(jnp.int32))
